# baseline (speedup 1.0000x reference)
import functools

import jax
import jax.numpy as jnp
from jax import lax
from jax.experimental import pallas as pl
from jax.experimental.pallas import tpu as pltpu
from jax.experimental.pallas import tpu_sc as plsc

_EPS = 1e-6
_K = 32
_JB = 512


def _sz(z):
    return jnp.where(z >= 0, z + _EPS, z - _EPS)


def _fwd1_body(x_ref, w_ref, b_ref, h_ref, zrow_ref):
    z = jax.lax.dot_general(
        x_ref[...], w_ref[...], (((1,), (1,)), ((), ())),
        preferred_element_type=jnp.float32) + b_ref[...]
    h_ref[...] = jnp.maximum(z, 0.0)
    ones_row = jnp.ones((1, w_ref.shape[1]), jnp.float32)
    zrow_ref[...] = jax.lax.dot_general(
        ones_row, w_ref[...], (((1,), (1,)), ((), ())),
        preferred_element_type=jnp.float32) + b_ref[...]


def _fwd2_body(a_ref, w_ref, b_ref, z_ref, h_ref):
    z = jax.lax.dot_general(
        a_ref[...], w_ref[...], (((1,), (1,)), ((), ())),
        preferred_element_type=jnp.float32) + b_ref[...]
    z_ref[...] = z
    h_ref[...] = jnp.maximum(z, 0.0)


def _fwd3_body(a_ref, w_ref, b_ref, h_ref):
    h_ref[...] = jax.lax.dot_general(
        a_ref[...], w_ref[...], (((1,), (1,)), ((), ())),
        preferred_element_type=jnp.float32) + b_ref[...]


def _topk_body(h3_ref, tk_ref, idx_ref, sv_ref):
    h3 = h3_ref[...]
    m = jnp.max(h3, axis=-1, keepdims=True)
    e = jnp.exp(h3 - m)
    r = e / jnp.sum(e, axis=-1, keepdims=True)
    iota = jax.lax.broadcasted_iota(jnp.int32, h3.shape, 1)
    g = r / _sz(h3)
    tk = tk_ref[0, 0]
    work = r
    idx_cols = []
    sv_cols = []
    for k in range(_K):
        cur = jnp.max(work, axis=-1, keepdims=True)
        sel = jnp.min(jnp.where(work == cur, iota, h3.shape[-1]),
                      axis=-1, keepdims=True)
        onehot = iota == sel
        gsel = jnp.sum(jnp.where(onehot, g, 0.0), axis=-1, keepdims=True)
        idx_cols.append(sel)
        sv_cols.append(jnp.broadcast_to(jnp.where(k < tk, gsel, 0.0),
                                        (gsel.shape[0], 16)))
        work = jnp.where(onehot, -1.0, work)
    idx_ref[...] = jnp.concatenate(idx_cols, axis=1)
    sv_ref[...] = jnp.concatenate(sv_cols, axis=1)


_SC_NC = 2
_SC_NS = 16
_SC_CH = 8


def _make_sc_bwd3(B, D):
    NW = _SC_NC * _SC_NS
    rows_per_w = B // NW
    nch = _K // _SC_CH
    nsl = D // 16
    mesh = plsc.VectorSubcoreMesh(core_axis_name="c", subcore_axis_name="s")

    @functools.partial(
        pl.kernel, mesh=mesh,
        out_type=jax.ShapeDtypeStruct((B, D), jnp.float32),
        scratch_types=[
            pltpu.VMEM((_K,), jnp.int32),
            pltpu.VMEM((_K * 16,), jnp.float32),
            pltpu.VMEM((2, _SC_CH, D), jnp.float32),
            pltpu.VMEM((D,), jnp.float32),
            pltpu.VMEM((D,), jnp.float32),
            pltpu.VMEM((D,), jnp.float32),
            pltpu.SemaphoreType.DMA,
            pltpu.SemaphoreType.DMA,
        ],
    )
    def sc_bwd3(w3_hbm, idx_hbm, sv_hbm, h2_hbm, z2_hbm, out_hbm,
                idx_v, sv_v, rows_v, acc_v, h2_v, z2_v, gsem, rsem):
        wid = lax.axis_index("s") * _SC_NC + lax.axis_index("c")
        for rr in range(rows_per_w):
            b = wid * rows_per_w + rr
            pltpu.sync_copy(idx_hbm.at[b], idx_v)
            pltpu.sync_copy(sv_hbm.at[b], sv_v)
            h2cp = pltpu.async_copy(h2_hbm.at[b], h2_v, rsem)
            z2cp = pltpu.async_copy(z2_hbm.at[b], z2_v, rsem)

            wgt = [sv_v[pl.ds(r * 16, 16)] for r in range(_K)]

            def start(c):
                return pltpu.async_copy(
                    w3_hbm.at[idx_v.at[pl.ds(c * _SC_CH, _SC_CH)]],
                    rows_v.at[c % 2], gsem)

            dma = start(0)
            for c in range(nch):
                nxt = start(c + 1) if c + 1 < nch else None
                dma.wait()

                def acc_body(j, _, c=c):
                    sl = pl.ds(j * 16, 16)
                    a = acc_v[sl] if c > 0 else jnp.zeros((16,), jnp.float32)
                    for r in range(_SC_CH):
                        a = a + wgt[c * _SC_CH + r] * rows_v[c % 2, r, sl]
                    acc_v[sl] = a
                    return 0

                lax.fori_loop(0, nsl, acc_body, 0, unroll=False)
                dma = nxt

            h2cp.wait()
            z2cp.wait()

            def epi_body(j, _):
                sl = pl.ds(j * 16, 16)
                acc_v[sl] = h2_v[sl] * acc_v[sl] / _sz(z2_v[sl])
                return 0

            lax.fori_loop(0, nsl, epi_body, 0, unroll=False)
            pltpu.sync_copy(acc_v, out_hbm.at[b])

    return sc_bwd3


def _bwd2_body(s2_ref, w_ref, h1_ref, zrow_ref, s1_ref):
    c = jax.lax.dot_general(
        s2_ref[...], w_ref[...], (((1,), (0,)), ((), ())),
        preferred_element_type=jnp.float32)
    s1_ref[...] = h1_ref[...] * c / _sz(zrow_ref[...])


def _bwd1_body(s1_ref, w_ref, out_ref):
    out_ref[...] = jax.lax.dot_general(
        s1_ref[...], w_ref[...], (((1,), (0,)), ((), ())),
        preferred_element_type=jnp.float32)


def _full(b, d):
    return pl.BlockSpec((b, d), lambda j: (0, 0))


def _colblk(b):
    return pl.BlockSpec((b, _JB), lambda j: (0, j))


def kernel(x, topk, W1, b1, W2, b2, W3, b3):
    B, D = x.shape
    grid = (D // _JB,)
    f32 = jnp.float32
    b1_2d, b2_2d, b3_2d = b1[None, :], b2[None, :], b3[None, :]
    tk = jnp.asarray(topk, jnp.int32).reshape(1, 1)

    wrow_spec = pl.BlockSpec((_JB, D), lambda j: (j, 0))
    wcol_spec = pl.BlockSpec((D, _JB), lambda j: (0, j))

    h1, z1row = pl.pallas_call(
        _fwd1_body, grid=grid,
        in_specs=[_full(B, D), wrow_spec, _colblk(1)],
        out_specs=[_colblk(B), _colblk(1)],
        out_shape=[jax.ShapeDtypeStruct((B, D), f32),
                   jax.ShapeDtypeStruct((1, D), f32)],
    )(x, W1, b1_2d)

    z2, h2 = pl.pallas_call(
        _fwd2_body, grid=grid,
        in_specs=[_full(B, D), wrow_spec, _colblk(1)],
        out_specs=[_colblk(B), _colblk(B)],
        out_shape=[jax.ShapeDtypeStruct((B, D), f32),
                   jax.ShapeDtypeStruct((B, D), f32)],
    )(h1, W2, b2_2d)

    h3 = pl.pallas_call(
        _fwd3_body, grid=grid,
        in_specs=[_full(B, D), wrow_spec, _colblk(1)],
        out_specs=_colblk(B),
        out_shape=jax.ShapeDtypeStruct((B, D), f32),
    )(h2, W3, b3_2d)

    idx, sv = pl.pallas_call(
        _topk_body,
        out_shape=[jax.ShapeDtypeStruct((B, _K), jnp.int32),
                   jax.ShapeDtypeStruct((B, _K * 16), f32)],
    )(h3, tk)

    s2 = _make_sc_bwd3(B, D)(W3, idx, sv, h2, z2)

    s1 = pl.pallas_call(
        _bwd2_body, grid=grid,
        in_specs=[_full(B, D), wcol_spec, _colblk(B), _colblk(1)],
        out_specs=_colblk(B),
        out_shape=jax.ShapeDtypeStruct((B, D), f32),
    )(s2, W2, h1, z1row)

    out = pl.pallas_call(
        _bwd1_body, grid=grid,
        in_specs=[_full(B, D), wcol_spec],
        out_specs=_colblk(B),
        out_shape=jax.ShapeDtypeStruct((B, D), f32),
    )(s1, W1)

    return out


# speedup vs baseline: 1.0007x; 1.0007x over previous
import functools

import jax
import jax.numpy as jnp
from jax import lax
from jax.experimental import pallas as pl
from jax.experimental.pallas import tpu as pltpu
from jax.experimental.pallas import tpu_sc as plsc

_EPS = 1e-6
_K = 32
_JB = 512


def _sz(z):
    return jnp.where(z >= 0, z + _EPS, z - _EPS)


def _fwd1_body(x_ref, w_ref, b_ref, h_ref, zrow_ref):
    z = jax.lax.dot_general(
        x_ref[...], w_ref[...], (((1,), (1,)), ((), ())),
        preferred_element_type=jnp.float32) + b_ref[...]
    h_ref[...] = jnp.maximum(z, 0.0)
    ones_row = jnp.ones((1, w_ref.shape[1]), jnp.float32)
    zrow_ref[...] = jax.lax.dot_general(
        ones_row, w_ref[...], (((1,), (1,)), ((), ())),
        preferred_element_type=jnp.float32) + b_ref[...]


def _fwd2_body(a_ref, w_ref, b_ref, z_ref, h_ref):
    z = jax.lax.dot_general(
        a_ref[...], w_ref[...], (((1,), (1,)), ((), ())),
        preferred_element_type=jnp.float32) + b_ref[...]
    z_ref[...] = z
    h_ref[...] = jnp.maximum(z, 0.0)


def _fwd3_body(a_ref, w_ref, b_ref, h_ref):
    h_ref[...] = jax.lax.dot_general(
        a_ref[...], w_ref[...], (((1,), (1,)), ((), ())),
        preferred_element_type=jnp.float32) + b_ref[...]


def _topk_body(h3_ref, tk_ref, idx_ref, sv_ref):
    h3 = h3_ref[...]
    m = jnp.max(h3, axis=-1, keepdims=True)
    e = jnp.exp(h3 - m)
    r = e / jnp.sum(e, axis=-1, keepdims=True)
    iota = jax.lax.broadcasted_iota(jnp.int32, h3.shape, 1)
    g = r / _sz(h3)
    tk = tk_ref[0, 0]
    work = r
    idx_cols = []
    sv_cols = []
    for k in range(_K):
        cur = jnp.max(work, axis=-1, keepdims=True)
        sel = jnp.min(jnp.where(work == cur, iota, h3.shape[-1]),
                      axis=-1, keepdims=True)
        onehot = iota == sel
        gsel = jnp.sum(jnp.where(onehot, g, 0.0), axis=-1, keepdims=True)
        idx_cols.append(sel)
        sv_cols.append(jnp.broadcast_to(jnp.where(k < tk, gsel, 0.0),
                                        (gsel.shape[0], 16)))
        work = jnp.where(onehot, -1.0, work)
    idx_ref[...] = jnp.concatenate(idx_cols, axis=1)
    sv_ref[...] = jnp.concatenate(sv_cols, axis=1)


_SC_NC = 2
_SC_NS = 16
_SC_CH = 8


def _make_sc_bwd3(B, D):
    NW = _SC_NC * _SC_NS
    rows_per_w = B // NW
    nch = _K // _SC_CH
    nsl = D // 16
    mesh = plsc.VectorSubcoreMesh(core_axis_name="c", subcore_axis_name="s")

    @functools.partial(
        pl.kernel, mesh=mesh,
        out_type=jax.ShapeDtypeStruct((B, D), jnp.float32),
        scratch_types=[
            pltpu.VMEM((rows_per_w, _K), jnp.int32),
            pltpu.VMEM((rows_per_w, _K * 16), jnp.float32),
            pltpu.VMEM((2, _SC_CH, D), jnp.float32),
            pltpu.VMEM((rows_per_w, D), jnp.float32),
            pltpu.VMEM((rows_per_w, D), jnp.float32),
            pltpu.VMEM((rows_per_w, D), jnp.float32),
            pltpu.SemaphoreType.DMA,
            pltpu.SemaphoreType.DMA,
            pltpu.SemaphoreType.DMA,
            pltpu.SemaphoreType.DMA,
        ],
    )
    def sc_bwd3(w3_hbm, idx_hbm, sv_hbm, h2_hbm, z2_hbm, out_hbm,
                idx_v, sv_v, rows_v, acc_v, h2_v, z2_v,
                gsem, isem, rsem, wsem):
        wid = lax.axis_index("s") * _SC_NC + lax.axis_index("c")

        icps, rcps = [], []
        for rr in range(rows_per_w):
            b = wid * rows_per_w + rr
            icps.append(pltpu.async_copy(idx_hbm.at[b], idx_v.at[rr], isem))
            icps.append(pltpu.async_copy(sv_hbm.at[b], sv_v.at[rr], isem))
            rcps.append(pltpu.async_copy(h2_hbm.at[b], h2_v.at[rr], rsem))
            rcps.append(pltpu.async_copy(z2_hbm.at[b], z2_v.at[rr], rsem))
        for cp in icps:
            cp.wait()

        def start(gc):
            rr, c = divmod(gc, nch)
            return pltpu.async_copy(
                w3_hbm.at[idx_v.at[rr, pl.ds(c * _SC_CH, _SC_CH)]],
                rows_v.at[gc % 2], gsem)

        nglobal = rows_per_w * nch
        dma = start(0)
        outcps = []
        waited_r = False
        for gc in range(nglobal):
            rr, c = divmod(gc, nch)
            nxt = start(gc + 1) if gc + 1 < nglobal else None
            dma.wait()

            def acc_body(j, _, rr=rr, c=c, gc=gc):
                sl = pl.ds(j * 16, 16)
                a = (acc_v[rr, sl] if c > 0
                     else jnp.zeros((16,), jnp.float32))
                for r in range(_SC_CH):
                    a = a + (sv_v[rr, pl.ds((c * _SC_CH + r) * 16, 16)]
                             * rows_v[gc % 2, r, sl])
                acc_v[rr, sl] = a
                return 0

            lax.fori_loop(0, nsl, acc_body, 0, unroll=8)
            dma = nxt

            if c == nch - 1:
                if not waited_r:
                    for cp in rcps:
                        cp.wait()
                    waited_r = True

                def epi_body(j, _, rr=rr):
                    sl = pl.ds(j * 16, 16)
                    acc_v[rr, sl] = (h2_v[rr, sl] * acc_v[rr, sl]
                                     / _sz(z2_v[rr, sl]))
                    return 0

                lax.fori_loop(0, nsl, epi_body, 0, unroll=8)
                outcps.append(pltpu.async_copy(
                    acc_v.at[rr], out_hbm.at[wid * rows_per_w + rr], wsem))
        for cp in outcps:
            cp.wait()

    return sc_bwd3


def _bwd2_body(s2_ref, w_ref, h1_ref, zrow_ref, s1_ref):
    c = jax.lax.dot_general(
        s2_ref[...], w_ref[...], (((1,), (0,)), ((), ())),
        preferred_element_type=jnp.float32)
    s1_ref[...] = h1_ref[...] * c / _sz(zrow_ref[...])


def _bwd1_body(s1_ref, w_ref, out_ref):
    out_ref[...] = jax.lax.dot_general(
        s1_ref[...], w_ref[...], (((1,), (0,)), ((), ())),
        preferred_element_type=jnp.float32)


def _full(b, d):
    return pl.BlockSpec((b, d), lambda j: (0, 0))


def _colblk(b):
    return pl.BlockSpec((b, _JB), lambda j: (0, j))


def kernel(x, topk, W1, b1, W2, b2, W3, b3):
    B, D = x.shape
    grid = (D // _JB,)
    f32 = jnp.float32
    b1_2d, b2_2d, b3_2d = b1[None, :], b2[None, :], b3[None, :]
    tk = jnp.asarray(topk, jnp.int32).reshape(1, 1)

    wrow_spec = pl.BlockSpec((_JB, D), lambda j: (j, 0))
    wcol_spec = pl.BlockSpec((D, _JB), lambda j: (0, j))

    h1, z1row = pl.pallas_call(
        _fwd1_body, grid=grid,
        in_specs=[_full(B, D), wrow_spec, _colblk(1)],
        out_specs=[_colblk(B), _colblk(1)],
        out_shape=[jax.ShapeDtypeStruct((B, D), f32),
                   jax.ShapeDtypeStruct((1, D), f32)],
    )(x, W1, b1_2d)

    z2, h2 = pl.pallas_call(
        _fwd2_body, grid=grid,
        in_specs=[_full(B, D), wrow_spec, _colblk(1)],
        out_specs=[_colblk(B), _colblk(B)],
        out_shape=[jax.ShapeDtypeStruct((B, D), f32),
                   jax.ShapeDtypeStruct((B, D), f32)],
    )(h1, W2, b2_2d)

    h3 = pl.pallas_call(
        _fwd3_body, grid=grid,
        in_specs=[_full(B, D), wrow_spec, _colblk(1)],
        out_specs=_colblk(B),
        out_shape=jax.ShapeDtypeStruct((B, D), f32),
    )(h2, W3, b3_2d)

    idx, sv = pl.pallas_call(
        _topk_body,
        out_shape=[jax.ShapeDtypeStruct((B, _K), jnp.int32),
                   jax.ShapeDtypeStruct((B, _K * 16), f32)],
    )(h3, tk)

    s2 = _make_sc_bwd3(B, D)(W3, idx, sv, h2, z2)

    s1 = pl.pallas_call(
        _bwd2_body, grid=grid,
        in_specs=[_full(B, D), wcol_spec, _colblk(B), _colblk(1)],
        out_specs=_colblk(B),
        out_shape=jax.ShapeDtypeStruct((B, D), f32),
    )(s2, W2, h1, z1row)

    out = pl.pallas_call(
        _bwd1_body, grid=grid,
        in_specs=[_full(B, D), wcol_spec],
        out_specs=_colblk(B),
        out_shape=jax.ShapeDtypeStruct((B, D), f32),
    )(s1, W1)

    return out


# speedup vs baseline: 1.0075x; 1.0068x over previous
import functools

import jax
import jax.numpy as jnp
from jax import lax
from jax.experimental import pallas as pl
from jax.experimental.pallas import tpu as pltpu
from jax.experimental.pallas import tpu_sc as plsc

_EPS = 1e-6
_K = 32
_JB = 512


def _sz(z):
    return jnp.where(z >= 0, z + _EPS, z - _EPS)


def _fwd1_body(x_ref, w_ref, b_ref, h_ref, zrow_ref):
    z = jax.lax.dot_general(
        x_ref[...], w_ref[...], (((1,), (1,)), ((), ())),
        preferred_element_type=jnp.float32) + b_ref[...]
    h_ref[...] = jnp.maximum(z, 0.0)
    ones_row = jnp.ones((1, w_ref.shape[1]), jnp.float32)
    zrow_ref[...] = jax.lax.dot_general(
        ones_row, w_ref[...], (((1,), (1,)), ((), ())),
        preferred_element_type=jnp.float32) + b_ref[...]


def _fwd2_body(a_ref, w_ref, b_ref, z_ref, h_ref):
    z = jax.lax.dot_general(
        a_ref[...], w_ref[...], (((1,), (1,)), ((), ())),
        preferred_element_type=jnp.float32) + b_ref[...]
    z_ref[...] = z
    h_ref[...] = jnp.maximum(z, 0.0)


def _fwd3_body(a_ref, w_ref, b_ref, h_ref):
    h_ref[...] = jax.lax.dot_general(
        a_ref[...], w_ref[...], (((1,), (1,)), ((), ())),
        preferred_element_type=jnp.float32) + b_ref[...]


def _topk_body(h3_ref, tk_ref, idx_ref, sv_ref):
    h3 = h3_ref[...]
    m = jnp.max(h3, axis=-1, keepdims=True)
    e = jnp.exp(h3 - m)
    r = e / jnp.sum(e, axis=-1, keepdims=True)
    iota = jax.lax.broadcasted_iota(jnp.int32, h3.shape, 1)
    g = r / _sz(h3)
    tk = tk_ref[0, 0]
    work = r
    idx_cols = []
    sv_cols = []
    for k in range(_K):
        cur = jnp.max(work, axis=-1, keepdims=True)
        sel = jnp.min(jnp.where(work == cur, iota, h3.shape[-1]),
                      axis=-1, keepdims=True)
        onehot = iota == sel
        gsel = jnp.sum(jnp.where(onehot, g, 0.0), axis=-1, keepdims=True)
        idx_cols.append(sel)
        sv_cols.append(jnp.broadcast_to(jnp.where(k < tk, gsel, 0.0),
                                        (gsel.shape[0], 16)))
        work = jnp.where(onehot, -1.0, work)
    idx_ref[...] = jnp.concatenate(idx_cols, axis=1)
    sv_ref[...] = jnp.concatenate(sv_cols, axis=1)


_SC_NC = 2
_SC_NS = 16
_SC_CH = 8


def _make_sc_bwd3(B, D):
    NW = _SC_NC * _SC_NS
    rows_per_w = B // NW
    nch = _K // _SC_CH
    nsl = D // 16
    mesh = plsc.VectorSubcoreMesh(core_axis_name="c", subcore_axis_name="s")

    @functools.partial(
        pl.kernel, mesh=mesh,
        out_type=jax.ShapeDtypeStruct((B, D), jnp.float32),
        scratch_types=[
            pltpu.VMEM((rows_per_w, _K), jnp.int32),
            pltpu.VMEM((rows_per_w, _K * 16), jnp.float32),
            pltpu.VMEM((2, _SC_CH, D), jnp.float32),
            pltpu.VMEM((rows_per_w, D), jnp.float32),
            pltpu.VMEM((rows_per_w, D), jnp.float32),
            pltpu.VMEM((rows_per_w, D), jnp.float32),
            pltpu.SemaphoreType.DMA,
            pltpu.SemaphoreType.DMA,
            pltpu.SemaphoreType.DMA,
            pltpu.SemaphoreType.DMA,
        ],
    )
    def sc_bwd3(w3_hbm, idx_hbm, sv_hbm, h2_hbm, z2_hbm, out_hbm,
                idx_v, sv_v, rows_v, acc_v, h2_v, z2_v,
                gsem, isem, rsem, wsem):
        wid = lax.axis_index("s") * _SC_NC + lax.axis_index("c")

        icps, rcps = [], []
        for rr in range(rows_per_w):
            b = wid * rows_per_w + rr
            icps.append(pltpu.async_copy(idx_hbm.at[b], idx_v.at[rr], isem))
            rcps.append(pltpu.async_copy(sv_hbm.at[b], sv_v.at[rr], rsem))
            rcps.append(pltpu.async_copy(h2_hbm.at[b], h2_v.at[rr], rsem))
            rcps.append(pltpu.async_copy(z2_hbm.at[b], z2_v.at[rr], rsem))
        for cp in icps:
            cp.wait()

        def start(gc):
            rr, c = divmod(gc, nch)
            return pltpu.async_copy(
                w3_hbm.at[idx_v.at[rr, pl.ds(c * _SC_CH, _SC_CH)]],
                rows_v.at[gc % 2], gsem)

        nglobal = rows_per_w * nch
        dma = start(0)
        outcps = []
        for cp in rcps:
            cp.wait()
        for gc in range(nglobal):
            rr, c = divmod(gc, nch)
            nxt = start(gc + 1) if gc + 1 < nglobal else None
            dma.wait()

            wgt = [sv_v[rr, pl.ds((c * _SC_CH + r) * 16, 16)]
                   for r in range(_SC_CH)]

            def acc_body(j, _, rr=rr, c=c, gc=gc, wgt=wgt):
                sl = pl.ds(j * 16, 16)
                a = (acc_v[rr, sl] if c > 0
                     else jnp.zeros((16,), jnp.float32))
                for r in range(_SC_CH):
                    a = a + wgt[r] * rows_v[gc % 2, r, sl]
                acc_v[rr, sl] = a
                return 0

            lax.fori_loop(0, nsl, acc_body, 0, unroll=8)
            dma = nxt

            if c == nch - 1:

                def epi_body(j, _, rr=rr):
                    sl = pl.ds(j * 16, 16)
                    acc_v[rr, sl] = (h2_v[rr, sl] * acc_v[rr, sl]
                                     / _sz(z2_v[rr, sl]))
                    return 0

                lax.fori_loop(0, nsl, epi_body, 0, unroll=8)
                outcps.append(pltpu.async_copy(
                    acc_v.at[rr], out_hbm.at[wid * rows_per_w + rr], wsem))
        for cp in outcps:
            cp.wait()

    return sc_bwd3


def _bwd2_body(s2_ref, w_ref, h1_ref, zrow_ref, s1_ref):
    c = jax.lax.dot_general(
        s2_ref[...], w_ref[...], (((1,), (0,)), ((), ())),
        preferred_element_type=jnp.float32)
    s1_ref[...] = h1_ref[...] * c / _sz(zrow_ref[...])


def _bwd1_body(s1_ref, w_ref, out_ref):
    out_ref[...] = jax.lax.dot_general(
        s1_ref[...], w_ref[...], (((1,), (0,)), ((), ())),
        preferred_element_type=jnp.float32)


def _full(b, d):
    return pl.BlockSpec((b, d), lambda j: (0, 0))


def _colblk(b):
    return pl.BlockSpec((b, _JB), lambda j: (0, j))


def kernel(x, topk, W1, b1, W2, b2, W3, b3):
    B, D = x.shape
    grid = (D // _JB,)
    f32 = jnp.float32
    b1_2d, b2_2d, b3_2d = b1[None, :], b2[None, :], b3[None, :]
    tk = jnp.asarray(topk, jnp.int32).reshape(1, 1)

    wrow_spec = pl.BlockSpec((_JB, D), lambda j: (j, 0))
    wcol_spec = pl.BlockSpec((D, _JB), lambda j: (0, j))

    h1, z1row = pl.pallas_call(
        _fwd1_body, grid=grid,
        in_specs=[_full(B, D), wrow_spec, _colblk(1)],
        out_specs=[_colblk(B), _colblk(1)],
        out_shape=[jax.ShapeDtypeStruct((B, D), f32),
                   jax.ShapeDtypeStruct((1, D), f32)],
    )(x, W1, b1_2d)

    z2, h2 = pl.pallas_call(
        _fwd2_body, grid=grid,
        in_specs=[_full(B, D), wrow_spec, _colblk(1)],
        out_specs=[_colblk(B), _colblk(B)],
        out_shape=[jax.ShapeDtypeStruct((B, D), f32),
                   jax.ShapeDtypeStruct((B, D), f32)],
    )(h1, W2, b2_2d)

    h3 = pl.pallas_call(
        _fwd3_body, grid=grid,
        in_specs=[_full(B, D), wrow_spec, _colblk(1)],
        out_specs=_colblk(B),
        out_shape=jax.ShapeDtypeStruct((B, D), f32),
    )(h2, W3, b3_2d)

    idx, sv = pl.pallas_call(
        _topk_body,
        out_shape=[jax.ShapeDtypeStruct((B, _K), jnp.int32),
                   jax.ShapeDtypeStruct((B, _K * 16), f32)],
    )(h3, tk)

    s2 = _make_sc_bwd3(B, D)(W3, idx, sv, h2, z2)

    s1 = pl.pallas_call(
        _bwd2_body, grid=grid,
        in_specs=[_full(B, D), wcol_spec, _colblk(B), _colblk(1)],
        out_specs=_colblk(B),
        out_shape=jax.ShapeDtypeStruct((B, D), f32),
    )(s2, W2, h1, z1row)

    out = pl.pallas_call(
        _bwd1_body, grid=grid,
        in_specs=[_full(B, D), wcol_spec],
        out_specs=_colblk(B),
        out_shape=jax.ShapeDtypeStruct((B, D), f32),
    )(s1, W1)

    return out


# speedup vs baseline: 1.0475x; 1.0397x over previous
import functools

import jax
import jax.numpy as jnp
from jax import lax
from jax.experimental import pallas as pl
from jax.experimental.pallas import tpu as pltpu
from jax.experimental.pallas import tpu_sc as plsc

_EPS = 1e-6
_K = 32
_JB = 512


def _sz(z):
    return jnp.where(z >= 0, z + _EPS, z - _EPS)


def _fwd1_body(x_ref, w_ref, b_ref, h_ref, zrow_ref):
    z = jax.lax.dot_general(
        x_ref[...], w_ref[...], (((1,), (1,)), ((), ())),
        preferred_element_type=jnp.float32) + b_ref[...]
    h_ref[...] = jnp.maximum(z, 0.0)
    ones_row = jnp.ones((1, w_ref.shape[1]), jnp.float32)
    zrow_ref[...] = jax.lax.dot_general(
        ones_row, w_ref[...], (((1,), (1,)), ((), ())),
        preferred_element_type=jnp.float32) + b_ref[...]


def _fwd2_body(a_ref, w_ref, b_ref, z_ref, h_ref):
    z = jax.lax.dot_general(
        a_ref[...], w_ref[...], (((1,), (1,)), ((), ())),
        preferred_element_type=jnp.float32) + b_ref[...]
    z_ref[...] = z
    h_ref[...] = jnp.maximum(z, 0.0)


def _fwd3_body(a_ref, w_ref, b_ref, h_ref):
    h_ref[...] = jax.lax.dot_general(
        a_ref[...], w_ref[...], (((1,), (1,)), ((), ())),
        preferred_element_type=jnp.float32) + b_ref[...]


def _topk_body(h3_ref, tk_ref, idx_ref, sv_ref):
    h3 = h3_ref[...]
    m = jnp.max(h3, axis=-1, keepdims=True)
    e = jnp.exp(h3 - m)
    r = e / jnp.sum(e, axis=-1, keepdims=True)
    iota = jax.lax.broadcasted_iota(jnp.int32, h3.shape, 1)
    g = r / _sz(h3)
    tk = tk_ref[0, 0]
    work = r
    idx_cols = []
    sv_cols = []
    for k in range(_K):
        cur = jnp.max(work, axis=-1, keepdims=True)
        sel = jnp.min(jnp.where(work == cur, iota, h3.shape[-1]),
                      axis=-1, keepdims=True)
        onehot = iota == sel
        gsel = jnp.sum(jnp.where(onehot, g, 0.0), axis=-1, keepdims=True)
        idx_cols.append(sel)
        sv_cols.append(jnp.broadcast_to(jnp.where(k < tk, gsel, 0.0),
                                        (gsel.shape[0], 16)))
        work = jnp.where(onehot, -1.0, work)
    idx_ref[...] = jnp.concatenate(idx_cols, axis=1)
    sv_ref[...] = jnp.concatenate(sv_cols, axis=1)


def _densify_body(idx_ref, sv_ref, s3_ref):
    n, d = s3_ref.shape
    iota = jax.lax.broadcasted_iota(jnp.int32, (n, d), 1)
    s3 = jnp.zeros((n, d), jnp.float32)
    for k in range(_K):
        s3 = s3 + jnp.where(iota == idx_ref[:, k:k + 1],
                            sv_ref[:, 16 * k:16 * k + 1], 0.0)
    s3_ref[...] = s3


def _bwd3_body(s3_ref, w_ref, h2_ref, z2_ref, s2_ref):
    c = jax.lax.dot_general(
        s3_ref[...], w_ref[...], (((1,), (0,)), ((), ())),
        preferred_element_type=jnp.float32)
    s2_ref[...] = h2_ref[...] * c / _sz(z2_ref[...])


_SC_NC = 2
_SC_NS = 16
_SC_CH = 8


def _make_sc_bwd3(B, D):
    NW = _SC_NC * _SC_NS
    rows_per_w = B // NW
    nch = _K // _SC_CH
    nsl = D // 16
    mesh = plsc.VectorSubcoreMesh(core_axis_name="c", subcore_axis_name="s")

    @functools.partial(
        pl.kernel, mesh=mesh,
        out_type=jax.ShapeDtypeStruct((B, D), jnp.float32),
        scratch_types=[
            pltpu.VMEM((rows_per_w, _K), jnp.int32),
            pltpu.VMEM((rows_per_w, _K * 16), jnp.float32),
            pltpu.VMEM((2, _SC_CH, D), jnp.float32),
            pltpu.VMEM((rows_per_w, D), jnp.float32),
            pltpu.VMEM((rows_per_w, D), jnp.float32),
            pltpu.VMEM((rows_per_w, D), jnp.float32),
            pltpu.SemaphoreType.DMA,
            pltpu.SemaphoreType.DMA,
            pltpu.SemaphoreType.DMA,
            pltpu.SemaphoreType.DMA,
        ],
    )
    def sc_bwd3(w3_hbm, idx_hbm, sv_hbm, h2_hbm, z2_hbm, out_hbm,
                idx_v, sv_v, rows_v, acc_v, h2_v, z2_v,
                gsem, isem, rsem, wsem):
        wid = lax.axis_index("s") * _SC_NC + lax.axis_index("c")

        icps, rcps = [], []
        for rr in range(rows_per_w):
            b = wid * rows_per_w + rr
            icps.append(pltpu.async_copy(idx_hbm.at[b], idx_v.at[rr], isem))
            rcps.append(pltpu.async_copy(sv_hbm.at[b], sv_v.at[rr], rsem))
            rcps.append(pltpu.async_copy(h2_hbm.at[b], h2_v.at[rr], rsem))
            rcps.append(pltpu.async_copy(z2_hbm.at[b], z2_v.at[rr], rsem))
        for cp in icps:
            cp.wait()

        def start(gc):
            rr, c = divmod(gc, nch)
            return pltpu.async_copy(
                w3_hbm.at[idx_v.at[rr, pl.ds(c * _SC_CH, _SC_CH)]],
                rows_v.at[gc % 2], gsem)

        nglobal = rows_per_w * nch
        dma = start(0)
        outcps = []
        for cp in rcps:
            cp.wait()
        for gc in range(nglobal):
            rr, c = divmod(gc, nch)
            nxt = start(gc + 1) if gc + 1 < nglobal else None
            dma.wait()

            wgt = [sv_v[rr, pl.ds((c * _SC_CH + r) * 16, 16)]
                   for r in range(_SC_CH)]

            def acc_body(j, _, rr=rr, c=c, gc=gc, wgt=wgt):
                sl = pl.ds(j * 16, 16)
                a = (acc_v[rr, sl] if c > 0
                     else jnp.zeros((16,), jnp.float32))
                for r in range(_SC_CH):
                    a = a + wgt[r] * rows_v[gc % 2, r, sl]
                acc_v[rr, sl] = a
                return 0

            lax.fori_loop(0, nsl, acc_body, 0, unroll=8)
            dma = nxt

            if c == nch - 1:

                def epi_body(j, _, rr=rr):
                    sl = pl.ds(j * 16, 16)
                    acc_v[rr, sl] = (h2_v[rr, sl] * acc_v[rr, sl]
                                     / _sz(z2_v[rr, sl]))
                    return 0

                lax.fori_loop(0, nsl, epi_body, 0, unroll=8)
                outcps.append(pltpu.async_copy(
                    acc_v.at[rr], out_hbm.at[wid * rows_per_w + rr], wsem))
        for cp in outcps:
            cp.wait()

    return sc_bwd3


def _bwd2_body(s2a_ref, s2b_ref, w_ref, h1_ref, zrow_ref, s1_ref):
    ca = jax.lax.dot_general(
        s2a_ref[...], w_ref[...], (((1,), (0,)), ((), ())),
        preferred_element_type=jnp.float32)
    cb = jax.lax.dot_general(
        s2b_ref[...], w_ref[...], (((1,), (0,)), ((), ())),
        preferred_element_type=jnp.float32)
    c = jnp.concatenate([ca, cb], axis=0)
    s1_ref[...] = h1_ref[...] * c / _sz(zrow_ref[...])


def _bwd1_body(s1_ref, w_ref, out_ref):
    out_ref[...] = jax.lax.dot_general(
        s1_ref[...], w_ref[...], (((1,), (0,)), ((), ())),
        preferred_element_type=jnp.float32)


def _full(b, d):
    return pl.BlockSpec((b, d), lambda j: (0, 0))


def _colblk(b):
    return pl.BlockSpec((b, _JB), lambda j: (0, j))


def kernel(x, topk, W1, b1, W2, b2, W3, b3):
    B, D = x.shape
    grid = (D // _JB,)
    f32 = jnp.float32
    b1_2d, b2_2d, b3_2d = b1[None, :], b2[None, :], b3[None, :]
    tk = jnp.asarray(topk, jnp.int32).reshape(1, 1)

    wrow_spec = pl.BlockSpec((_JB, D), lambda j: (j, 0))
    wcol_spec = pl.BlockSpec((D, _JB), lambda j: (0, j))

    h1, z1row = pl.pallas_call(
        _fwd1_body, grid=grid,
        in_specs=[_full(B, D), wrow_spec, _colblk(1)],
        out_specs=[_colblk(B), _colblk(1)],
        out_shape=[jax.ShapeDtypeStruct((B, D), f32),
                   jax.ShapeDtypeStruct((1, D), f32)],
    )(x, W1, b1_2d)

    z2, h2 = pl.pallas_call(
        _fwd2_body, grid=grid,
        in_specs=[_full(B, D), wrow_spec, _colblk(1)],
        out_specs=[_colblk(B), _colblk(B)],
        out_shape=[jax.ShapeDtypeStruct((B, D), f32),
                   jax.ShapeDtypeStruct((B, D), f32)],
    )(h1, W2, b2_2d)

    h3 = pl.pallas_call(
        _fwd3_body, grid=grid,
        in_specs=[_full(B, D), wrow_spec, _colblk(1)],
        out_specs=_colblk(B),
        out_shape=jax.ShapeDtypeStruct((B, D), f32),
    )(h2, W3, b3_2d)

    idx, sv = pl.pallas_call(
        _topk_body,
        out_shape=[jax.ShapeDtypeStruct((B, _K), jnp.int32),
                   jax.ShapeDtypeStruct((B, _K * 16), f32)],
    )(h3, tk)

    Bh = B // 2
    s2a = _make_sc_bwd3(Bh, D)(W3, idx, sv, h2, z2)

    s3b = pl.pallas_call(
        _densify_body, grid=(1,),
        in_specs=[pl.BlockSpec((Bh, _K), lambda i: (1, 0)),
                  pl.BlockSpec((Bh, _K * 16), lambda i: (1, 0))],
        out_specs=pl.BlockSpec((Bh, D), lambda i: (0, 0)),
        out_shape=jax.ShapeDtypeStruct((Bh, D), f32),
    )(idx, sv)

    s2b = pl.pallas_call(
        _bwd3_body, grid=grid,
        in_specs=[_full(Bh, D), wcol_spec,
                  pl.BlockSpec((Bh, _JB), lambda j: (1, j)),
                  pl.BlockSpec((Bh, _JB), lambda j: (1, j))],
        out_specs=_colblk(Bh),
        out_shape=jax.ShapeDtypeStruct((Bh, D), f32),
    )(s3b, W3, h2, z2)

    s1 = pl.pallas_call(
        _bwd2_body, grid=grid,
        in_specs=[_full(Bh, D), _full(Bh, D), wcol_spec,
                  _colblk(B), _colblk(1)],
        out_specs=_colblk(B),
        out_shape=jax.ShapeDtypeStruct((B, D), f32),
    )(s2a, s2b, W2, h1, z1row)

    out = pl.pallas_call(
        _bwd1_body, grid=grid,
        in_specs=[_full(B, D), wcol_spec],
        out_specs=_colblk(B),
        out_shape=jax.ShapeDtypeStruct((B, D), f32),
    )(s1, W1)

    return out


# speedup vs baseline: 1.0557x; 1.0078x over previous
import functools

import jax
import jax.numpy as jnp
from jax import lax
from jax.experimental import pallas as pl
from jax.experimental.pallas import tpu as pltpu
from jax.experimental.pallas import tpu_sc as plsc

_EPS = 1e-6
_K = 32
_JB = 512


def _sz(z):
    return jnp.where(z >= 0, z + _EPS, z - _EPS)


def _fwd1_body(x_ref, w_ref, b_ref, h_ref, zrow_ref):
    z = jax.lax.dot_general(
        x_ref[...], w_ref[...], (((1,), (1,)), ((), ())),
        preferred_element_type=jnp.float32) + b_ref[...]
    h_ref[...] = jnp.maximum(z, 0.0)
    ones_row = jnp.ones((1, w_ref.shape[1]), jnp.float32)
    zrow_ref[...] = jax.lax.dot_general(
        ones_row, w_ref[...], (((1,), (1,)), ((), ())),
        preferred_element_type=jnp.float32) + b_ref[...]


def _fwd2_body(a_ref, w_ref, b_ref, z_ref, h_ref):
    z = jax.lax.dot_general(
        a_ref[...], w_ref[...], (((1,), (1,)), ((), ())),
        preferred_element_type=jnp.float32) + b_ref[...]
    z_ref[...] = z
    h_ref[...] = jnp.maximum(z, 0.0)


def _fwd3_body(a_ref, w_ref, b_ref, h_ref):
    h_ref[...] = jax.lax.dot_general(
        a_ref[...], w_ref[...], (((1,), (1,)), ((), ())),
        preferred_element_type=jnp.float32) + b_ref[...]


def _topk_body(h3_ref, tk_ref, idx_ref, sv_ref):
    h3 = h3_ref[...]
    m = jnp.max(h3, axis=-1, keepdims=True)
    e = jnp.exp(h3 - m)
    r = e / jnp.sum(e, axis=-1, keepdims=True)
    iota = jax.lax.broadcasted_iota(jnp.int32, h3.shape, 1)
    g = r / _sz(h3)
    tk = tk_ref[0, 0]
    work = r
    idx_cols = []
    sv_cols = []
    for k in range(_K):
        cur = jnp.max(work, axis=-1, keepdims=True)
        sel = jnp.min(jnp.where(work == cur, iota, h3.shape[-1]),
                      axis=-1, keepdims=True)
        onehot = iota == sel
        gsel = jnp.sum(jnp.where(onehot, g, 0.0), axis=-1, keepdims=True)
        idx_cols.append(sel)
        sv_cols.append(jnp.broadcast_to(jnp.where(k < tk, gsel, 0.0),
                                        (gsel.shape[0], 16)))
        work = jnp.where(onehot, -1.0, work)
    idx_ref[...] = jnp.concatenate(idx_cols, axis=1)
    sv_ref[...] = jnp.concatenate(sv_cols, axis=1)


def _densify_body(idx_ref, sv_ref, s3_ref):
    n, d = s3_ref.shape
    iota = jax.lax.broadcasted_iota(jnp.int32, (n, d), 1)
    s3 = jnp.zeros((n, d), jnp.float32)
    for k in range(_K):
        s3 = s3 + jnp.where(iota == idx_ref[:, k:k + 1],
                            sv_ref[:, 16 * k:16 * k + 1], 0.0)
    s3_ref[...] = s3


def _bwd3_body(s3_ref, w_ref, h2_ref, z2_ref, s2_ref):
    c = jax.lax.dot_general(
        s3_ref[...], w_ref[...], (((1,), (0,)), ((), ())),
        preferred_element_type=jnp.float32)
    s2_ref[...] = h2_ref[...] * c / _sz(z2_ref[...])


_SC_NC = 2
_SC_NS = 16
_SC_CH = 8


def _make_sc_bwd3(B, D):
    NW = _SC_NC * _SC_NS
    rows_per_w = B // NW
    nch = _K // _SC_CH
    nsl = D // 16
    mesh = plsc.VectorSubcoreMesh(core_axis_name="c", subcore_axis_name="s")

    @functools.partial(
        pl.kernel, mesh=mesh,
        out_type=jax.ShapeDtypeStruct((B, D), jnp.float32),
        scratch_types=[
            pltpu.VMEM((rows_per_w, _K), jnp.int32),
            pltpu.VMEM((rows_per_w, _K * 16), jnp.float32),
            pltpu.VMEM((2, _SC_CH, D), jnp.float32),
            pltpu.VMEM((rows_per_w, D), jnp.float32),
            pltpu.SemaphoreType.DMA,
            pltpu.SemaphoreType.DMA,
            pltpu.SemaphoreType.DMA,
            pltpu.SemaphoreType.DMA,
        ],
    )
    def sc_bwd3(w3_hbm, idx_hbm, sv_hbm, out_hbm,
                idx_v, sv_v, rows_v, acc_v, gsem, isem, rsem, wsem):
        wid = lax.axis_index("s") * _SC_NC + lax.axis_index("c")

        icps, rcps = [], []
        for rr in range(rows_per_w):
            b = wid * rows_per_w + rr
            icps.append(pltpu.async_copy(idx_hbm.at[b], idx_v.at[rr], isem))
            rcps.append(pltpu.async_copy(sv_hbm.at[b], sv_v.at[rr], rsem))
        for cp in icps:
            cp.wait()

        def start(gc):
            rr, c = divmod(gc, nch)
            return pltpu.async_copy(
                w3_hbm.at[idx_v.at[rr, pl.ds(c * _SC_CH, _SC_CH)]],
                rows_v.at[gc % 2], gsem)

        nglobal = rows_per_w * nch
        dma = start(0)
        outcps = []
        for cp in rcps:
            cp.wait()
        for gc in range(nglobal):
            rr, c = divmod(gc, nch)
            nxt = start(gc + 1) if gc + 1 < nglobal else None
            dma.wait()

            wgt = [sv_v[rr, pl.ds((c * _SC_CH + r) * 16, 16)]
                   for r in range(_SC_CH)]

            def acc_body(j, _, rr=rr, c=c, gc=gc, wgt=wgt):
                sl = pl.ds(j * 16, 16)
                a = (acc_v[rr, sl] if c > 0
                     else jnp.zeros((16,), jnp.float32))
                for r in range(_SC_CH):
                    a = a + wgt[r] * rows_v[gc % 2, r, sl]
                acc_v[rr, sl] = a
                return 0

            lax.fori_loop(0, nsl, acc_body, 0, unroll=8)
            dma = nxt

            if c == nch - 1:
                outcps.append(pltpu.async_copy(
                    acc_v.at[rr], out_hbm.at[wid * rows_per_w + rr], wsem))
        for cp in outcps:
            cp.wait()

    return sc_bwd3


def _bwd2_body(c3a_ref, s2b_ref, h2a_ref, z2a_ref, w_ref, h1_ref,
               zrow_ref, s1_ref):
    s2a = h2a_ref[...] * c3a_ref[...] / _sz(z2a_ref[...])
    ca = jax.lax.dot_general(
        s2a, w_ref[...], (((1,), (0,)), ((), ())),
        preferred_element_type=jnp.float32)
    cb = jax.lax.dot_general(
        s2b_ref[...], w_ref[...], (((1,), (0,)), ((), ())),
        preferred_element_type=jnp.float32)
    c = jnp.concatenate([ca, cb], axis=0)
    s1_ref[...] = h1_ref[...] * c / _sz(zrow_ref[...])


def _bwd1_body(s1_ref, w_ref, out_ref):
    out_ref[...] = jax.lax.dot_general(
        s1_ref[...], w_ref[...], (((1,), (0,)), ((), ())),
        preferred_element_type=jnp.float32)


def _full(b, d):
    return pl.BlockSpec((b, d), lambda j: (0, 0))


def _colblk(b):
    return pl.BlockSpec((b, _JB), lambda j: (0, j))


def kernel(x, topk, W1, b1, W2, b2, W3, b3):
    B, D = x.shape
    grid = (D // _JB,)
    f32 = jnp.float32
    b1_2d, b2_2d, b3_2d = b1[None, :], b2[None, :], b3[None, :]
    tk = jnp.asarray(topk, jnp.int32).reshape(1, 1)

    wrow_spec = pl.BlockSpec((_JB, D), lambda j: (j, 0))
    wcol_spec = pl.BlockSpec((D, _JB), lambda j: (0, j))

    h1, z1row = pl.pallas_call(
        _fwd1_body, grid=grid,
        in_specs=[_full(B, D), wrow_spec, _colblk(1)],
        out_specs=[_colblk(B), _colblk(1)],
        out_shape=[jax.ShapeDtypeStruct((B, D), f32),
                   jax.ShapeDtypeStruct((1, D), f32)],
    )(x, W1, b1_2d)

    z2, h2 = pl.pallas_call(
        _fwd2_body, grid=grid,
        in_specs=[_full(B, D), wrow_spec, _colblk(1)],
        out_specs=[_colblk(B), _colblk(B)],
        out_shape=[jax.ShapeDtypeStruct((B, D), f32),
                   jax.ShapeDtypeStruct((B, D), f32)],
    )(h1, W2, b2_2d)

    h3 = pl.pallas_call(
        _fwd3_body, grid=grid,
        in_specs=[_full(B, D), wrow_spec, _colblk(1)],
        out_specs=_colblk(B),
        out_shape=jax.ShapeDtypeStruct((B, D), f32),
    )(h2, W3, b3_2d)

    idx, sv = pl.pallas_call(
        _topk_body,
        out_shape=[jax.ShapeDtypeStruct((B, _K), jnp.int32),
                   jax.ShapeDtypeStruct((B, _K * 16), f32)],
    )(h3, tk)

    Bh = B // 2
    c3a = _make_sc_bwd3(Bh, D)(W3, idx, sv)

    s3b = pl.pallas_call(
        _densify_body, grid=(1,),
        in_specs=[pl.BlockSpec((Bh, _K), lambda i: (1, 0)),
                  pl.BlockSpec((Bh, _K * 16), lambda i: (1, 0))],
        out_specs=pl.BlockSpec((Bh, D), lambda i: (0, 0)),
        out_shape=jax.ShapeDtypeStruct((Bh, D), f32),
    )(idx, sv)

    s2b = pl.pallas_call(
        _bwd3_body, grid=grid,
        in_specs=[_full(Bh, D), wcol_spec,
                  pl.BlockSpec((Bh, _JB), lambda j: (1, j)),
                  pl.BlockSpec((Bh, _JB), lambda j: (1, j))],
        out_specs=_colblk(Bh),
        out_shape=jax.ShapeDtypeStruct((Bh, D), f32),
    )(s3b, W3, h2, z2)

    half_spec = pl.BlockSpec((Bh, D), lambda j: (0, 0))
    s1 = pl.pallas_call(
        _bwd2_body, grid=grid,
        in_specs=[_full(Bh, D), _full(Bh, D), half_spec, half_spec,
                  wcol_spec, _colblk(B), _colblk(1)],
        out_specs=_colblk(B),
        out_shape=jax.ShapeDtypeStruct((B, D), f32),
    )(c3a, s2b, h2, z2, W2, h1, z1row)

    out = pl.pallas_call(
        _bwd1_body, grid=grid,
        in_specs=[_full(B, D), wcol_spec],
        out_specs=_colblk(B),
        out_shape=jax.ShapeDtypeStruct((B, D), f32),
    )(s1, W1)

    return out


# speedup vs baseline: 1.0589x; 1.0031x over previous
import functools

import jax
import jax.numpy as jnp
from jax import lax
from jax.experimental import pallas as pl
from jax.experimental.pallas import tpu as pltpu
from jax.experimental.pallas import tpu_sc as plsc

_EPS = 1e-6
_K = 32
_JB = 512


def _sz(z):
    return jnp.where(z >= 0, z + _EPS, z - _EPS)


def _fwd1_body(x_ref, w_ref, b_ref, h_ref, zrow_ref):
    z = jax.lax.dot_general(
        x_ref[...], w_ref[...], (((1,), (1,)), ((), ())),
        preferred_element_type=jnp.float32) + b_ref[...]
    h_ref[...] = jnp.maximum(z, 0.0)
    ones_row = jnp.ones((1, w_ref.shape[1]), jnp.float32)
    zrow_ref[...] = jax.lax.dot_general(
        ones_row, w_ref[...], (((1,), (1,)), ((), ())),
        preferred_element_type=jnp.float32) + b_ref[...]


def _fwd2_body(a_ref, w_ref, b_ref, z_ref, h_ref):
    z = jax.lax.dot_general(
        a_ref[...], w_ref[...], (((1,), (1,)), ((), ())),
        preferred_element_type=jnp.float32) + b_ref[...]
    z_ref[...] = z
    h_ref[...] = jnp.maximum(z, 0.0)


def _fwd3_body(a_ref, w_ref, b_ref, h_ref):
    h_ref[...] = jax.lax.dot_general(
        a_ref[...], w_ref[...], (((1,), (1,)), ((), ())),
        preferred_element_type=jnp.float32) + b_ref[...]


def _topk_body(h3_ref, tk_ref, idx_ref, sv_ref, s3b_ref):
    h3 = h3_ref[...]
    bh = s3b_ref.shape[0]
    m = jnp.max(h3, axis=-1, keepdims=True)
    e = jnp.exp(h3 - m)
    r = e / jnp.sum(e, axis=-1, keepdims=True)
    iota = jax.lax.broadcasted_iota(jnp.int32, h3.shape, 1)
    g = r / _sz(h3)
    tk = tk_ref[0, 0]
    work = r
    idx_cols = []
    sv_cols = []
    s3b = jnp.zeros((bh, h3.shape[-1]), jnp.float32)
    for k in range(_K):
        cur = jnp.max(work, axis=-1, keepdims=True)
        sel = jnp.min(jnp.where(work == cur, iota, h3.shape[-1]),
                      axis=-1, keepdims=True)
        onehot = iota == sel
        gsel = jnp.sum(jnp.where(onehot, g, 0.0), axis=-1, keepdims=True)
        keep = jnp.where(k < tk, gsel, 0.0)
        idx_cols.append(sel[:bh])
        sv_cols.append(jnp.broadcast_to(keep[:bh], (bh, 16)))
        s3b = s3b + jnp.where(onehot[bh:], keep[bh:], 0.0)
        work = jnp.where(onehot, -1.0, work)
    idx_ref[...] = jnp.concatenate(idx_cols, axis=1)
    sv_ref[...] = jnp.concatenate(sv_cols, axis=1)
    s3b_ref[...] = s3b


def _bwd3_body(s3_ref, w_ref, h2_ref, z2_ref, s2_ref):
    c = jax.lax.dot_general(
        s3_ref[...], w_ref[...], (((1,), (0,)), ((), ())),
        preferred_element_type=jnp.float32)
    s2_ref[...] = h2_ref[...] * c / _sz(z2_ref[...])


_SC_NC = 2
_SC_NS = 16
_SC_CH = 8


def _make_sc_bwd3(B, D):
    NW = _SC_NC * _SC_NS
    rows_per_w = B // NW
    nch = _K // _SC_CH
    nsl = D // 16
    mesh = plsc.VectorSubcoreMesh(core_axis_name="c", subcore_axis_name="s")

    @functools.partial(
        pl.kernel, mesh=mesh,
        out_type=jax.ShapeDtypeStruct((B, D), jnp.float32),
        scratch_types=[
            pltpu.VMEM((rows_per_w, _K), jnp.int32),
            pltpu.VMEM((rows_per_w, _K * 16), jnp.float32),
            pltpu.VMEM((2, _SC_CH, D), jnp.float32),
            pltpu.VMEM((rows_per_w, D), jnp.float32),
            pltpu.SemaphoreType.DMA,
            pltpu.SemaphoreType.DMA,
            pltpu.SemaphoreType.DMA,
            pltpu.SemaphoreType.DMA,
        ],
    )
    def sc_bwd3(w3_hbm, idx_hbm, sv_hbm, out_hbm,
                idx_v, sv_v, rows_v, acc_v, gsem, isem, rsem, wsem):
        wid = lax.axis_index("s") * _SC_NC + lax.axis_index("c")

        icps, rcps = [], []
        for rr in range(rows_per_w):
            b = wid * rows_per_w + rr
            icps.append(pltpu.async_copy(idx_hbm.at[b], idx_v.at[rr], isem))
            rcps.append(pltpu.async_copy(sv_hbm.at[b], sv_v.at[rr], rsem))
        for cp in icps:
            cp.wait()

        def start(gc):
            rr, c = divmod(gc, nch)
            return pltpu.async_copy(
                w3_hbm.at[idx_v.at[rr, pl.ds(c * _SC_CH, _SC_CH)]],
                rows_v.at[gc % 2], gsem)

        nglobal = rows_per_w * nch
        dma = start(0)
        outcps = []
        for cp in rcps:
            cp.wait()
        for gc in range(nglobal):
            rr, c = divmod(gc, nch)
            nxt = start(gc + 1) if gc + 1 < nglobal else None
            dma.wait()

            wgt = [sv_v[rr, pl.ds((c * _SC_CH + r) * 16, 16)]
                   for r in range(_SC_CH)]

            def acc_body(j, _, rr=rr, c=c, gc=gc, wgt=wgt):
                sl = pl.ds(j * 16, 16)
                a = (acc_v[rr, sl] if c > 0
                     else jnp.zeros((16,), jnp.float32))
                for r in range(_SC_CH):
                    a = a + wgt[r] * rows_v[gc % 2, r, sl]
                acc_v[rr, sl] = a
                return 0

            lax.fori_loop(0, nsl, acc_body, 0, unroll=8)
            dma = nxt

            if c == nch - 1:
                outcps.append(pltpu.async_copy(
                    acc_v.at[rr], out_hbm.at[wid * rows_per_w + rr], wsem))
        for cp in outcps:
            cp.wait()

    return sc_bwd3


def _bwd2_body(c3a_ref, s2b_ref, h2a_ref, z2a_ref, w_ref, h1_ref,
               zrow_ref, s1_ref):
    s2a = h2a_ref[...] * c3a_ref[...] / _sz(z2a_ref[...])
    ca = jax.lax.dot_general(
        s2a, w_ref[...], (((1,), (0,)), ((), ())),
        preferred_element_type=jnp.float32)
    cb = jax.lax.dot_general(
        s2b_ref[...], w_ref[...], (((1,), (0,)), ((), ())),
        preferred_element_type=jnp.float32)
    c = jnp.concatenate([ca, cb], axis=0)
    s1_ref[...] = h1_ref[...] * c / _sz(zrow_ref[...])


def _bwd1_body(s1_ref, w_ref, out_ref):
    out_ref[...] = jax.lax.dot_general(
        s1_ref[...], w_ref[...], (((1,), (0,)), ((), ())),
        preferred_element_type=jnp.float32)


def _full(b, d):
    return pl.BlockSpec((b, d), lambda j: (0, 0))


def _colblk(b):
    return pl.BlockSpec((b, _JB), lambda j: (0, j))


def kernel(x, topk, W1, b1, W2, b2, W3, b3):
    B, D = x.shape
    grid = (D // _JB,)
    f32 = jnp.float32
    b1_2d, b2_2d, b3_2d = b1[None, :], b2[None, :], b3[None, :]
    tk = jnp.asarray(topk, jnp.int32).reshape(1, 1)

    wrow_spec = pl.BlockSpec((_JB, D), lambda j: (j, 0))
    wcol_spec = pl.BlockSpec((D, _JB), lambda j: (0, j))

    h1, z1row = pl.pallas_call(
        _fwd1_body, grid=grid,
        in_specs=[_full(B, D), wrow_spec, _colblk(1)],
        out_specs=[_colblk(B), _colblk(1)],
        out_shape=[jax.ShapeDtypeStruct((B, D), f32),
                   jax.ShapeDtypeStruct((1, D), f32)],
    )(x, W1, b1_2d)

    z2, h2 = pl.pallas_call(
        _fwd2_body, grid=grid,
        in_specs=[_full(B, D), wrow_spec, _colblk(1)],
        out_specs=[_colblk(B), _colblk(B)],
        out_shape=[jax.ShapeDtypeStruct((B, D), f32),
                   jax.ShapeDtypeStruct((B, D), f32)],
    )(h1, W2, b2_2d)

    h3 = pl.pallas_call(
        _fwd3_body, grid=grid,
        in_specs=[_full(B, D), wrow_spec, _colblk(1)],
        out_specs=_colblk(B),
        out_shape=jax.ShapeDtypeStruct((B, D), f32),
    )(h2, W3, b3_2d)

    Bh = B // 2
    idx, sv, s3b = pl.pallas_call(
        _topk_body,
        out_shape=[jax.ShapeDtypeStruct((Bh, _K), jnp.int32),
                   jax.ShapeDtypeStruct((Bh, _K * 16), f32),
                   jax.ShapeDtypeStruct((Bh, D), f32)],
    )(h3, tk)

    c3a = _make_sc_bwd3(Bh, D)(W3, idx, sv)

    s2b = pl.pallas_call(
        _bwd3_body, grid=grid,
        in_specs=[_full(Bh, D), wcol_spec,
                  pl.BlockSpec((Bh, _JB), lambda j: (1, j)),
                  pl.BlockSpec((Bh, _JB), lambda j: (1, j))],
        out_specs=_colblk(Bh),
        out_shape=jax.ShapeDtypeStruct((Bh, D), f32),
    )(s3b, W3, h2, z2)

    half_spec = pl.BlockSpec((Bh, D), lambda j: (0, 0))
    s1 = pl.pallas_call(
        _bwd2_body, grid=grid,
        in_specs=[_full(Bh, D), _full(Bh, D), half_spec, half_spec,
                  wcol_spec, _colblk(B), _colblk(1)],
        out_specs=_colblk(B),
        out_shape=jax.ShapeDtypeStruct((B, D), f32),
    )(c3a, s2b, h2, z2, W2, h1, z1row)

    out = pl.pallas_call(
        _bwd1_body, grid=grid,
        in_specs=[_full(B, D), wcol_spec],
        out_specs=_colblk(B),
        out_shape=jax.ShapeDtypeStruct((B, D), f32),
    )(s1, W1)

    return out
